# Initial kernel scaffold; baseline (speedup 1.0000x reference)
import jax
import jax.numpy as jnp
from jax.experimental import pallas as pl


def kernel(x, table):
    raise NotImplementedError("write your pallas kernel here")


# speedup vs baseline: 8.7239x; 8.7239x over previous
import functools

import jax
import jax.numpy as jnp
from jax import lax
from jax.experimental import pallas as pl
from jax.experimental.pallas import tpu as pltpu
from jax.experimental.pallas import tpu_sc as plsc

_OFFSET_RANGE = 100
_LANES = 16
_NC = 2
_NS = 16
_NW = _NC * _NS


_NBUF = 4


def _sc_gather(n_rows_per_w: int, n_total: int, d: int):
    mesh = plsc.VectorSubcoreMesh(core_axis_name="c", subcore_axis_name="s")
    n_rounds = n_rows_per_w // _NBUF

    @functools.partial(
        pl.kernel,
        mesh=mesh,
        out_type=jax.ShapeDtypeStruct((n_total, d), jnp.float32),
        scratch_types=[
            pltpu.VMEM((n_rows_per_w, 128), jnp.int32),
            pltpu.VMEM((n_rows_per_w, 128), jnp.int32),
        ]
        + [pltpu.VMEM((128, d), jnp.float32) for _ in range(_NBUF)]
        + [pltpu.SemaphoreType.DMA for _ in range(2 * _NBUF)],
    )
    def body(x_hbm, off_hbm, table_hbm, out_hbm, idx_v, off_v, *bufs_and_sems):
        rows = bufs_and_sems[:_NBUF]
        sem_g = bufs_and_sems[_NBUF:2 * _NBUF]
        sem_o = bufs_and_sems[2 * _NBUF:]
        wid = lax.axis_index("s") * _NC + lax.axis_index("c")
        rbase = wid * n_rows_per_w

        pltpu.sync_copy(x_hbm.at[pl.ds(rbase, n_rows_per_w)], idx_v)
        pltpu.sync_copy(off_hbm.at[pl.ds(rbase, n_rows_per_w)], off_v)

        def add_row(j, carry):
            for k in range(128 // _LANES):
                sl = pl.ds(k * _LANES, _LANES)
                idx_v[j, sl] = idx_v[j, sl] + off_v[j, sl]
            return carry

        lax.fori_loop(0, n_rows_per_w, add_row, 0)

        def gather_start(j, b):
            pltpu.async_copy(table_hbm.at[idx_v.at[j]], rows[b], sem_g[b])

        def gather_wait(b):
            pltpu.make_async_copy(
                table_hbm.at[pl.ds(0, 128)], rows[b], sem_g[b]
            ).wait()

        def out_copy(j, b):
            dst = out_hbm.at[pl.ds((rbase + j) * 128, 128)]
            return pltpu.async_copy(rows[b], dst, sem_o[b])

        for b in range(_NBUF):
            gather_start(b, b)

        def round_body(g, carry):
            j0 = g * _NBUF
            for b in range(_NBUF):
                gather_wait(b)
                out_copy(j0 + b, b).wait()
                gather_start(j0 + b + _NBUF, b)
            return carry

        lax.fori_loop(0, n_rounds - 1, round_body, 0)

        j0 = (n_rounds - 1) * _NBUF
        for b in range(_NBUF):
            gather_wait(b)
            out_copy(j0 + b, b).wait()

    return body


def kernel(x, table):
    batch, hist = x.shape
    n = batch * hist
    d = table.shape[1]

    offset = jax.random.randint(
        jax.random.key(42), (batch, 1), 1, _OFFSET_RANGE + 1
    )
    off_full = jnp.broadcast_to(offset, (batch, hist)).astype(jnp.int32)

    n_rows = n // 128
    x2 = x.astype(jnp.int32).reshape(n_rows, 128)
    off2 = off_full.reshape(n_rows, 128)

    out = _sc_gather(n_rows // _NW, n, d)(x2, off2, table)
    return out.reshape(batch, hist, d)
